# baseline (speedup 1.0000x reference)
import functools

import jax
import jax.numpy as jnp
from jax import lax
from jax.experimental import pallas as pl
from jax.experimental.pallas import tpu as pltpu
from jax.experimental.pallas import tpu_sc as plsc

N = 10000
E = 320000
F = 128
H = 128
C = 16

NC = 2
NS = 16
NW = NC * NS
K = 128
NCHUNK = 80
EPAD = NW * NCHUNK * K
NPAD = 10240
RPT = NPAD // NS

_mesh = plsc.VectorSubcoreMesh(core_axis_name="c", subcore_axis_name="s")


def _deg_body(dst_hbm, ones_hbm, zeros_hbm, out_hbm, dstv, onesv, acc, sdeg):
    c = lax.axis_index("c")
    s = lax.axis_index("s")
    wid = s * NC + c
    pltpu.sync_copy(zeros_hbm.at[pl.ds(s * RPT, RPT)], acc.at[pl.ds(s * RPT, RPT)])
    pltpu.sync_copy(ones_hbm, onesv)
    pltpu.sync_copy(dst_hbm.at[wid], dstv)
    plsc.subcore_barrier()

    @pl.loop(0, NCHUNK, step=4)
    def _(j):
        for k in range(4):
            pltpu.async_copy(onesv, acc.at[dstv.at[j + k]], sdeg, add=True)
        for k in range(4):
            pltpu.make_async_copy(onesv, acc.at[dstv.at[j + k]], sdeg).wait()

    plsc.subcore_barrier()
    pltpu.sync_copy(acc.at[pl.ds(s * RPT, RPT)], out_hbm.at[c, pl.ds(s * RPT, RPT)])


NBLK = NCHUNK // 8


def _agg128_body(h_hbm, src_hbm, dst_hbm, zeros_hbm, out_hbm,
                 dstv, ib0, ib1, r0, r1, acc, sr0, sr1, sb0, sb1):
    c = lax.axis_index("c")
    s = lax.axis_index("s")
    wid = s * NC + c
    ib = (ib0, ib1)
    sb = (sb0, sb1)
    rr = (r0, r1)
    sr = (sr0, sr1)
    pltpu.sync_copy(zeros_hbm.at[pl.ds(s * RPT, RPT)], acc.at[pl.ds(s * RPT, RPT)])
    pltpu.sync_copy(dst_hbm.at[wid], dstv)
    pltpu.async_copy(src_hbm.at[wid, 0], ib0, sb0)
    pltpu.async_copy(src_hbm.at[wid, 1], ib1, sb1)
    plsc.subcore_barrier()
    pltpu.make_async_copy(src_hbm.at[wid, 0], ib0, sb0).wait()
    pltpu.async_copy(h_hbm.at[ib0.at[0]], r0, sr0)

    @pl.loop(0, NBLK, step=2)
    def _(j):
        for half in range(2):
            gb = j + half
            cur, nxt = ib[half], ib[1 - half]
            scur, snxt = sb[half], sb[1 - half]
            for k in range(8):
                ch = gb * 8 + k
                pltpu.make_async_copy(h_hbm.at[cur.at[k]], rr[k % 2], sr[k % 2]).wait()
                if k < 7:
                    pltpu.async_copy(h_hbm.at[cur.at[k + 1]], rr[(k + 1) % 2],
                                     sr[(k + 1) % 2])
                else:
                    @pl.when(gb + 1 < NBLK)
                    def _(gb=gb, nxt=nxt, snxt=snxt, k=k):
                        pltpu.make_async_copy(src_hbm.at[wid, gb + 1], nxt, snxt).wait()
                        pltpu.async_copy(h_hbm.at[nxt.at[0]], rr[(k + 1) % 2],
                                         sr[(k + 1) % 2])

                    @pl.when(gb + 2 < NBLK)
                    def _(gb=gb, cur=cur, scur=scur):
                        pltpu.async_copy(src_hbm.at[wid, gb + 2], cur, scur)

                pltpu.sync_copy(rr[k % 2], acc.at[dstv.at[ch]], add=True)

    plsc.subcore_barrier()
    pltpu.sync_copy(acc.at[pl.ds(s * RPT, RPT)], out_hbm.at[c, pl.ds(s * RPT, RPT)])


def _make_agg128():
    return pl.kernel(
        _agg128_body,
        out_type=jax.ShapeDtypeStruct((NC, NPAD, H), jnp.float32),
        mesh=_mesh,
        scratch_types=[
            pltpu.VMEM((NCHUNK, K), jnp.int32),
            pltpu.VMEM((8, K), jnp.int32),
            pltpu.VMEM((8, K), jnp.int32),
            pltpu.VMEM((K, H), jnp.float32),
            pltpu.VMEM((K, H), jnp.float32),
            pltpu.VMEM_SHARED((NPAD, H), jnp.float32),
            pltpu.SemaphoreType.DMA,
            pltpu.SemaphoreType.DMA,
            pltpu.SemaphoreType.DMA,
            pltpu.SemaphoreType.DMA,
        ],
    )


def _agg_body(Dw, h_hbm, src_hbm, dst_hbm, zeros_hbm, out_hbm,
              dstv, si0, si1, si2, si3, r0, r1, acc,
              sr0, sr1, ss0, ss1, ss2, ss3):
    c = lax.axis_index("c")
    s = lax.axis_index("s")
    wid = s * NC + c
    si = (si0, si1, si2, si3)
    ss = (ss0, ss1, ss2, ss3)
    rr = (r0, r1)
    sr = (sr0, sr1)
    pltpu.sync_copy(zeros_hbm.at[pl.ds(s * RPT, RPT)], acc.at[pl.ds(s * RPT, RPT)])
    pltpu.sync_copy(dst_hbm.at[wid], dstv)
    for b in range(4):
        pltpu.async_copy(src_hbm.at[wid, b], si[b], ss[b])
    plsc.subcore_barrier()
    pltpu.make_async_copy(src_hbm.at[wid, 0], si0, ss0).wait()
    pltpu.async_copy(h_hbm.at[si0], r0, sr0)

    @pl.loop(0, NCHUNK, step=4)
    def _(j):
        for k in range(4):
            ch = j + k
            kn = (k + 1) % 4

            pltpu.make_async_copy(h_hbm.at[si[k]], rr[k % 2], sr[k % 2]).wait()

            def _issue_next(kn=kn, ch=ch):
                pltpu.make_async_copy(src_hbm.at[wid, ch + 1], si[kn], ss[kn]).wait()
                pltpu.async_copy(h_hbm.at[si[kn]], rr[(k + 1) % 2], sr[(k + 1) % 2])

            if k < 3:
                _issue_next()
            else:
                pl.when(ch + 1 < NCHUNK)(_issue_next)

            pltpu.sync_copy(rr[k % 2], acc.at[dstv.at[ch]], add=True)

            @pl.when(ch + 4 < NCHUNK)
            def _(k=k, ch=ch):
                pltpu.async_copy(src_hbm.at[wid, ch + 4], si[k], ss[k])

    plsc.subcore_barrier()
    pltpu.sync_copy(acc.at[pl.ds(s * RPT, RPT)], out_hbm.at[c, pl.ds(s * RPT, RPT)])


def _make_deg():
    return pl.kernel(
        _deg_body,
        out_type=jax.ShapeDtypeStruct((NC, NPAD, 16), jnp.float32),
        mesh=_mesh,
        compiler_params=pltpu.CompilerParams(use_tc_tiling_on_sc=False),
        scratch_types=[
            pltpu.VMEM((NCHUNK, K), jnp.int32),
            pltpu.VMEM((K, 16), jnp.float32),
            pltpu.VMEM_SHARED((NPAD, 16), jnp.float32),
            pltpu.SemaphoreType.DMA,
        ],
    )


def _make_agg(Dw):
    return pl.kernel(
        functools.partial(_agg_body, Dw),
        out_type=jax.ShapeDtypeStruct((NC, NPAD, Dw), jnp.float32),
        mesh=_mesh,
        compiler_params=pltpu.CompilerParams(use_tc_tiling_on_sc=False),
        scratch_types=[
            pltpu.VMEM((NCHUNK, K), jnp.int32),
            pltpu.VMEM((K,), jnp.int32),
            pltpu.VMEM((K,), jnp.int32),
            pltpu.VMEM((K,), jnp.int32),
            pltpu.VMEM((K,), jnp.int32),
            pltpu.VMEM((K, Dw), jnp.float32),
            pltpu.VMEM((K, Dw), jnp.float32),
            pltpu.VMEM_SHARED((NPAD, Dw), jnp.float32),
            pltpu.SemaphoreType.DMA,
            pltpu.SemaphoreType.DMA,
            pltpu.SemaphoreType.DMA,
            pltpu.SemaphoreType.DMA,
            pltpu.SemaphoreType.DMA,
            pltpu.SemaphoreType.DMA,
        ],
    )


def _mm_body(x_ref, w_ref, o_ref):
    o_ref[...] = jnp.dot(x_ref[...], w_ref[...], preferred_element_type=jnp.float32)


def _norm_body(degp_ref, h1_ref, dinv_ref, h1s_ref):
    deg = degp_ref[0][:N, 0:1] + degp_ref[1][:N, 0:1] + 1.0
    dinv = lax.rsqrt(jnp.maximum(deg, 1.0))
    dinv_ref[...] = dinv
    h1s_ref[...] = h1_ref[...] * dinv


def _mid_body(agg_ref, h1s_ref, dinv_ref, b1_ref, w2_ref, h2s_ref):
    dinv = dinv_ref[...]
    agg = agg_ref[0][:N] + agg_ref[1][:N] + h1s_ref[...]
    z = jnp.maximum(agg * dinv + b1_ref[...], 0.0)
    h2s_ref[...] = jnp.dot(z * dinv, w2_ref[...], preferred_element_type=jnp.float32)


def _fin_body(agg_ref, h2s_ref, dinv_ref, b2_ref, out_ref):
    agg = agg_ref[0][:N] + agg_ref[1][:N] + h2s_ref[...]
    out_ref[...] = agg * dinv_ref[...] + b2_ref[...]


def kernel(x, edge_index, W1, b1, W2, b2):
    ei = edge_index.astype(jnp.int32)
    npad_e = EPAD - E
    src3 = jnp.concatenate([ei[0], jnp.zeros((npad_e,), jnp.int32)]).reshape(
        NW, NCHUNK, K)
    dst3 = jnp.concatenate([ei[1], jnp.full((npad_e,), N, jnp.int32)]).reshape(
        NW, NCHUNK, K)
    ones_k = jnp.ones((K, 16), jnp.float32)
    zeros16 = jnp.zeros((NPAD, 16), jnp.float32)
    zeros128 = jnp.zeros((NPAD, H), jnp.float32)

    degp = _make_deg()(dst3, ones_k, zeros16)

    h1 = pl.pallas_call(
        _mm_body, out_shape=jax.ShapeDtypeStruct((N, H), jnp.float32)
    )(x, W1)

    dinv, h1s = pl.pallas_call(
        _norm_body,
        out_shape=(
            jax.ShapeDtypeStruct((N, 1), jnp.float32),
            jax.ShapeDtypeStruct((N, H), jnp.float32),
        ),
    )(degp, h1)

    agg1 = _make_agg128()(h1s, src3.reshape(NW, NBLK, 8, K), dst3, zeros128)

    h2s = pl.pallas_call(
        _mid_body, out_shape=jax.ShapeDtypeStruct((N, C), jnp.float32)
    )(agg1, h1s, dinv, b1.reshape(1, H), W2)

    agg2 = _make_agg(C)(h2s, src3, dst3, zeros16)

    out = pl.pallas_call(
        _fin_body, out_shape=jax.ShapeDtypeStruct((N, C), jnp.float32)
    )(agg2, h2s, dinv, b2.reshape(1, C))
    return out


# speedup vs baseline: 1.0130x; 1.0130x over previous
import functools

import jax
import jax.numpy as jnp
from jax import lax
from jax.experimental import pallas as pl
from jax.experimental.pallas import tpu as pltpu
from jax.experimental.pallas import tpu_sc as plsc

N = 10000
E = 320000
F = 128
H = 128
C = 16

NC = 2
NS = 16
NW = NC * NS
K = 128
NCHUNK = 80
EPAD = NW * NCHUNK * K
NPAD = 10240
RPT = NPAD // NS

_mesh = plsc.VectorSubcoreMesh(core_axis_name="c", subcore_axis_name="s")


def _deg_body(dst_hbm, ones_hbm, zeros_hbm, out_hbm, dstv, onesv, acc, sdeg):
    c = lax.axis_index("c")
    s = lax.axis_index("s")
    wid = s * NC + c
    pltpu.sync_copy(zeros_hbm.at[pl.ds(s * RPT, RPT)], acc.at[pl.ds(s * RPT, RPT)])
    pltpu.sync_copy(ones_hbm, onesv)
    pltpu.sync_copy(dst_hbm.at[wid], dstv)
    plsc.subcore_barrier()

    @pl.loop(0, NCHUNK, step=4)
    def _(j):
        for k in range(4):
            pltpu.async_copy(onesv, acc.at[dstv.at[j + k]], sdeg, add=True)
        for k in range(4):
            pltpu.make_async_copy(onesv, acc.at[dstv.at[j + k]], sdeg).wait()

    plsc.subcore_barrier()
    pltpu.sync_copy(acc.at[pl.ds(s * RPT, RPT)], out_hbm.at[c, pl.ds(s * RPT, RPT)])


NBLK = NCHUNK // 8


def _agg128_body(h_hbm, src_hbm, dst_hbm, zeros_hbm, out_hbm,
                 dstv, ib0, ib1, r0, r1, acc, sr0, sr1, sb0, sb1):
    c = lax.axis_index("c")
    s = lax.axis_index("s")
    wid = s * NC + c
    ib = (ib0, ib1)
    sb = (sb0, sb1)
    rr = (r0, r1)
    sr = (sr0, sr1)
    pltpu.sync_copy(zeros_hbm.at[pl.ds(s * RPT, RPT)], acc.at[pl.ds(s * RPT, RPT)])
    pltpu.sync_copy(dst_hbm.at[wid], dstv)
    pltpu.async_copy(src_hbm.at[wid, 0], ib0, sb0)
    pltpu.async_copy(src_hbm.at[wid, 1], ib1, sb1)
    plsc.subcore_barrier()
    pltpu.make_async_copy(src_hbm.at[wid, 0], ib0, sb0).wait()
    pltpu.async_copy(h_hbm.at[ib0.at[0]], r0, sr0)

    @pl.loop(0, NBLK, step=2)
    def _(j):
        for half in range(2):
            gb = j + half
            cur, nxt = ib[half], ib[1 - half]
            scur, snxt = sb[half], sb[1 - half]
            for k in range(8):
                ch = gb * 8 + k
                pltpu.make_async_copy(h_hbm.at[cur.at[k]], rr[k % 2], sr[k % 2]).wait()
                if k < 7:
                    pltpu.async_copy(h_hbm.at[cur.at[k + 1]], rr[(k + 1) % 2],
                                     sr[(k + 1) % 2])
                else:
                    @pl.when(gb + 1 < NBLK)
                    def _(gb=gb, nxt=nxt, snxt=snxt, k=k):
                        pltpu.make_async_copy(src_hbm.at[wid, gb + 1], nxt, snxt).wait()
                        pltpu.async_copy(h_hbm.at[nxt.at[0]], rr[(k + 1) % 2],
                                         sr[(k + 1) % 2])

                    @pl.when(gb + 2 < NBLK)
                    def _(gb=gb, cur=cur, scur=scur):
                        pltpu.async_copy(src_hbm.at[wid, gb + 2], cur, scur)

                pltpu.sync_copy(rr[k % 2], acc.at[dstv.at[ch]], add=True)

    plsc.subcore_barrier()
    pltpu.sync_copy(acc.at[pl.ds(s * RPT, RPT)], out_hbm.at[c, pl.ds(s * RPT, RPT)])


def _make_agg128():
    return pl.kernel(
        _agg128_body,
        out_type=jax.ShapeDtypeStruct((NC, NPAD, H), jnp.float32),
        mesh=_mesh,
        scratch_types=[
            pltpu.VMEM((NCHUNK, K), jnp.int32),
            pltpu.VMEM((8, K), jnp.int32),
            pltpu.VMEM((8, K), jnp.int32),
            pltpu.VMEM((K, H), jnp.float32),
            pltpu.VMEM((K, H), jnp.float32),
            pltpu.VMEM_SHARED((NPAD, H), jnp.float32),
            pltpu.SemaphoreType.DMA,
            pltpu.SemaphoreType.DMA,
            pltpu.SemaphoreType.DMA,
            pltpu.SemaphoreType.DMA,
        ],
    )


def _agg_body(Dw, h_hbm, src_hbm, dst_hbm, zeros_hbm, out_hbm,
              dstv, si0, si1, si2, si3, r0, r1, acc,
              sr0, sr1, ss0, ss1, ss2, ss3):
    c = lax.axis_index("c")
    s = lax.axis_index("s")
    wid = s * NC + c
    si = (si0, si1, si2, si3)
    ss = (ss0, ss1, ss2, ss3)
    rr = (r0, r1)
    sr = (sr0, sr1)
    pltpu.sync_copy(zeros_hbm.at[pl.ds(s * RPT, RPT)], acc.at[pl.ds(s * RPT, RPT)])
    pltpu.sync_copy(dst_hbm.at[wid], dstv)
    for b in range(4):
        pltpu.async_copy(src_hbm.at[wid, b], si[b], ss[b])
    plsc.subcore_barrier()
    pltpu.make_async_copy(src_hbm.at[wid, 0], si0, ss0).wait()
    pltpu.async_copy(h_hbm.at[si0], r0, sr0)

    @pl.loop(0, NCHUNK, step=4)
    def _(j):
        for k in range(4):
            ch = j + k
            kn = (k + 1) % 4

            pltpu.make_async_copy(h_hbm.at[si[k]], rr[k % 2], sr[k % 2]).wait()

            def _issue_next(kn=kn, ch=ch):
                pltpu.make_async_copy(src_hbm.at[wid, ch + 1], si[kn], ss[kn]).wait()
                pltpu.async_copy(h_hbm.at[si[kn]], rr[(k + 1) % 2], sr[(k + 1) % 2])

            if k < 3:
                _issue_next()
            else:
                pl.when(ch + 1 < NCHUNK)(_issue_next)

            pltpu.sync_copy(rr[k % 2], acc.at[dstv.at[ch]], add=True)

            @pl.when(ch + 4 < NCHUNK)
            def _(k=k, ch=ch):
                pltpu.async_copy(src_hbm.at[wid, ch + 4], si[k], ss[k])

    plsc.subcore_barrier()
    pltpu.sync_copy(acc.at[pl.ds(s * RPT, RPT)], out_hbm.at[c, pl.ds(s * RPT, RPT)])


def _make_deg():
    return pl.kernel(
        _deg_body,
        out_type=jax.ShapeDtypeStruct((NC, NPAD, 16), jnp.float32),
        mesh=_mesh,
        compiler_params=pltpu.CompilerParams(use_tc_tiling_on_sc=False),
        scratch_types=[
            pltpu.VMEM((NCHUNK, K), jnp.int32),
            pltpu.VMEM((K, 16), jnp.float32),
            pltpu.VMEM_SHARED((NPAD, 16), jnp.float32),
            pltpu.SemaphoreType.DMA,
        ],
    )


def _make_agg(Dw):
    return pl.kernel(
        functools.partial(_agg_body, Dw),
        out_type=jax.ShapeDtypeStruct((NC, NPAD, Dw), jnp.float32),
        mesh=_mesh,
        compiler_params=pltpu.CompilerParams(use_tc_tiling_on_sc=False),
        scratch_types=[
            pltpu.VMEM((NCHUNK, K), jnp.int32),
            pltpu.VMEM((K,), jnp.int32),
            pltpu.VMEM((K,), jnp.int32),
            pltpu.VMEM((K,), jnp.int32),
            pltpu.VMEM((K,), jnp.int32),
            pltpu.VMEM((K, Dw), jnp.float32),
            pltpu.VMEM((K, Dw), jnp.float32),
            pltpu.VMEM_SHARED((NPAD, Dw), jnp.float32),
            pltpu.SemaphoreType.DMA,
            pltpu.SemaphoreType.DMA,
            pltpu.SemaphoreType.DMA,
            pltpu.SemaphoreType.DMA,
            pltpu.SemaphoreType.DMA,
            pltpu.SemaphoreType.DMA,
        ],
    )


def _mm_body(x_ref, w_ref, o_ref):
    o_ref[...] = jnp.dot(x_ref[...], w_ref[...], preferred_element_type=jnp.float32)


def _norm_body(degp_ref, h1_ref, dinv_ref, h1s_ref):
    deg = degp_ref[0][:N, 0:1] + degp_ref[1][:N, 0:1] + 1.0
    dinv = lax.rsqrt(jnp.maximum(deg, 1.0))
    dinv_ref[...] = dinv
    h1s_ref[...] = h1_ref[...] * dinv


def _mid_body(agg_ref, h1s_ref, dinv_ref, b1_ref, w2_ref, h2s_ref):
    dinv = dinv_ref[...]
    agg = agg_ref[0][:N] + agg_ref[1][:N] + h1s_ref[...]
    z = jnp.maximum(agg * dinv + b1_ref[...], 0.0)
    h2s_ref[...] = jnp.dot(z * dinv, w2_ref[...], preferred_element_type=jnp.float32)


def _fin_body(agg_ref, h2s_ref, dinv_ref, b2_ref, out_ref):
    agg = agg_ref[0][:N] + agg_ref[1][:N] + h2s_ref[...]
    out_ref[...] = agg * dinv_ref[...] + b2_ref[...]


def kernel(x, edge_index, W1, b1, W2, b2):
    ei = edge_index.astype(jnp.int32)
    npad_e = EPAD - E
    dummy_dst = N + (jnp.arange(npad_e, dtype=jnp.int32) % (NPAD - N))
    src3 = jnp.concatenate([ei[0], jnp.zeros((npad_e,), jnp.int32)]).reshape(
        NW, NCHUNK, K)
    dst3 = jnp.concatenate([ei[1], dummy_dst]).reshape(NW, NCHUNK, K)
    ones_k = jnp.ones((K, 16), jnp.float32)
    zeros16 = jnp.zeros((NPAD, 16), jnp.float32)
    zeros128 = jnp.zeros((NPAD, H), jnp.float32)

    degp = _make_deg()(dst3, ones_k, zeros16)

    h1 = pl.pallas_call(
        _mm_body, out_shape=jax.ShapeDtypeStruct((N, H), jnp.float32)
    )(x, W1)

    dinv, h1s = pl.pallas_call(
        _norm_body,
        out_shape=(
            jax.ShapeDtypeStruct((N, 1), jnp.float32),
            jax.ShapeDtypeStruct((N, H), jnp.float32),
        ),
    )(degp, h1)

    agg1 = _make_agg128()(h1s, src3.reshape(NW, NBLK, 8, K), dst3, zeros128)

    h2s = pl.pallas_call(
        _mid_body, out_shape=jax.ShapeDtypeStruct((N, C), jnp.float32)
    )(agg1, h1s, dinv, b1.reshape(1, H), W2)

    agg2 = _make_agg(C)(h2s, src3, dst3, zeros16)

    out = pl.pallas_call(
        _fin_body, out_shape=jax.ShapeDtypeStruct((N, C), jnp.float32)
    )(agg2, h2s, dinv, b2.reshape(1, C))
    return out
